# initial kernel scaffold (unmeasured)
import jax
import jax.numpy as jnp
from jax import lax
from jax.experimental import pallas as pl
from jax.experimental.pallas import tpu as pltpu

N_DEV = 4
S_LOC = 1024
H = 8
D = 128
BLK = 64
SCALE = 0.08838834764831843


def kernel(x, Wq, K_ext, V_ext, Wo):
    def body(x_ref, wq_ref, k_ref, v_ref, wo_ref, out_ref,
             kv_scr, send_sems, recv_sems):
        my = lax.axis_index("i")
        left = (my - 1) % N_DEV
        right = (my + 1) % N_DEV

        kv_scr[0, 0] = jnp.transpose(k_ref[0].astype(jnp.bfloat16), (1, 0, 2))
        kv_scr[0, 1] = jnp.transpose(v_ref[0].astype(jnp.bfloat16), (1, 0, 2))

        barrier_sem = pltpu.get_barrier_semaphore()
        for nbr in (left, right):
            pl.semaphore_signal(barrier_sem, inc=1, device_id=(nbr,),
                                device_id_type=pl.DeviceIdType.MESH)
        pl.semaphore_wait(barrier_sem, 2)

        q = jnp.dot(x_ref[0].astype(jnp.bfloat16), wq_ref[...].astype(jnp.bfloat16),
                    preferred_element_type=jnp.float32)
        q = (q * SCALE).astype(jnp.bfloat16)

        qb = (my * S_LOC + lax.broadcasted_iota(jnp.int32, (S_LOC, 1), 0)) // BLK

        acc = [jnp.zeros((S_LOC, D), jnp.float32) for _ in range(H)]
        wsum = [jnp.zeros((S_LOC, 1), jnp.float32) for _ in range(H)]

        for h in range(N_DEV):
            if h < N_DEV - 1:
                rdma = pltpu.make_async_remote_copy(
                    src_ref=kv_scr.at[h],
                    dst_ref=kv_scr.at[h + 1],
                    send_sem=send_sems.at[h],
                    recv_sem=recv_sems.at[h],
                    device_id=(right,),
                    device_id_type=pl.DeviceIdType.MESH,
                )
                rdma.start()

            origin = (my - h) % N_DEV
            kb = (origin * S_LOC
                  + lax.broadcasted_iota(jnp.int32, (1, S_LOC), 1)) // BLK
            mask = (qb == kb) | (kb == 0) | ((qb + kb) % 3 == 0)

            for hd in range(H):
                qh = q[:, hd * D:(hd + 1) * D]
                s = lax.dot_general(qh, kv_scr[h, 0, hd],
                                    (((1,), (1,)), ((), ())),
                                    preferred_element_type=jnp.float32)
                w = jnp.where(mask, jnp.exp(s), 0.0)
                wsum[hd] = wsum[hd] + jnp.sum(w, axis=1, keepdims=True)
                acc[hd] = acc[hd] + lax.dot_general(
                    w.astype(jnp.bfloat16), kv_scr[h, 1, hd],
                    (((1,), (0,)), ((), ())),
                    preferred_element_type=jnp.float32)

            if h < N_DEV - 1:
                rdma.wait()

        ctx = jnp.concatenate(
            [(acc[hd] / wsum[hd]).astype(jnp.bfloat16) for hd in range(H)],
            axis=1)
        out_ref[0] = jnp.dot(ctx, wo_ref[...].astype(jnp.bfloat16),
                             preferred_element_type=jnp.float32)

    return pl.pallas_call(
        body,
        out_shape=jax.ShapeDtypeStruct((1, S_LOC, H * D), jnp.float32),
        in_specs=[pl.BlockSpec(memory_space=pltpu.VMEM)] * 5,
        out_specs=pl.BlockSpec(memory_space=pltpu.VMEM),
        scratch_shapes=[
            pltpu.VMEM((N_DEV, 2, H, S_LOC, D), jnp.bfloat16),
            pltpu.SemaphoreType.DMA((N_DEV - 1,)),
            pltpu.SemaphoreType.DMA((N_DEV - 1,)),
        ],
        compiler_params=pltpu.CompilerParams(collective_id=0),
    )(x, Wq, K_ext, V_ext, Wo)


# baseline (device time: 185830 ns/iter reference)
import jax
import jax.numpy as jnp
from jax import lax
from jax.experimental import pallas as pl
from jax.experimental.pallas import tpu as pltpu

N_DEV = 4
S_LOC = 1024
H = 8
D = 128
BLK = 64
SCALE = 0.08838834764831843
BF = jnp.bfloat16


def kernel(x, Wq, K_ext, V_ext, Wo):
    def body(x_ref, wq_ref, k_ref, v_ref, wo_ref, out_ref,
             kv_scr, stage, q_scr, acc_scr, wsum_scr,
             copy_sem, send_sems, recv_sems):
        my = lax.axis_index("i")
        left = (my - 1) % N_DEV
        right = (my + 1) % N_DEV

        cp = pltpu.make_async_copy(k_ref, stage, copy_sem)
        cp.start()
        cp.wait()
        kv_scr[0, 0] = jnp.reshape(stage[0], (S_LOC, H * D)).astype(BF)
        cp = pltpu.make_async_copy(v_ref, stage, copy_sem)
        cp.start()
        cp.wait()
        kv_scr[0, 1] = jnp.reshape(stage[0], (S_LOC, H * D)).astype(BF)

        barrier_sem = pltpu.get_barrier_semaphore()
        for nbr in (left, right):
            pl.semaphore_signal(barrier_sem, inc=1, device_id=(nbr,),
                                device_id_type=pl.DeviceIdType.MESH)
        pl.semaphore_wait(barrier_sem, 2)

        q_scr[...] = (jnp.dot(x_ref[0].astype(BF), wq_ref[...].astype(BF),
                              preferred_element_type=jnp.float32)
                      * SCALE).astype(BF)

        acc_scr[...] = jnp.zeros((S_LOC, H * D), jnp.float32)
        wsum_scr[...] = jnp.zeros((S_LOC, H), jnp.float32)

        R = S_LOC // 2

        for h in range(N_DEV):
            slot = h % 3
            if h < N_DEV - 1:
                rdma = pltpu.make_async_remote_copy(
                    src_ref=kv_scr.at[slot],
                    dst_ref=kv_scr.at[(h + 1) % 3],
                    send_sem=send_sems.at[h],
                    recv_sem=recv_sems.at[h],
                    device_id=(right,),
                    device_id_type=pl.DeviceIdType.MESH,
                )
                rdma.start()

            origin = (my - h) % N_DEV
            kb = (origin * S_LOC
                  + lax.broadcasted_iota(jnp.int32, (1, S_LOC), 1)) // BLK
            for r in range(S_LOC // R):
                rs = slice(r * R, (r + 1) * R)
                qb = (my * S_LOC + r * R
                      + lax.broadcasted_iota(jnp.int32, (R, 1), 0)) // BLK
                mask = (qb == kb) | (kb == 0) | ((qb + kb) % 3 == 0)
                for hd in range(H):
                    sl = slice(hd * D, (hd + 1) * D)
                    s = lax.dot_general(q_scr[rs, sl], kv_scr[slot, 0, :, sl],
                                        (((1,), (1,)), ((), ())),
                                        preferred_element_type=jnp.float32)
                    w = jnp.where(mask, jnp.exp(s), 0.0)
                    wsum_scr[rs, hd:hd + 1] = (
                        wsum_scr[rs, hd:hd + 1]
                        + jnp.sum(w, axis=1, keepdims=True))
                    acc_scr[rs, sl] = acc_scr[rs, sl] + lax.dot_general(
                        w.astype(BF), kv_scr[slot, 1, :, sl],
                        (((1,), (0,)), ((), ())),
                        preferred_element_type=jnp.float32)

            if h < N_DEV - 1:
                rdma.wait()

        out = jnp.zeros((S_LOC, H * D), jnp.float32)
        for hd in range(H):
            sl = slice(hd * D, (hd + 1) * D)
            ctx = (acc_scr[:, sl] / wsum_scr[:, hd:hd + 1]).astype(BF)
            out = out + jnp.dot(ctx, wo_ref[sl, :].astype(BF),
                                preferred_element_type=jnp.float32)
        out_ref[0] = out

    return pl.pallas_call(
        body,
        out_shape=jax.ShapeDtypeStruct((1, S_LOC, H * D), jnp.float32),
        in_specs=[
            pl.BlockSpec(memory_space=pltpu.VMEM),
            pl.BlockSpec(memory_space=pltpu.VMEM),
            pl.BlockSpec(memory_space=pl.ANY),
            pl.BlockSpec(memory_space=pl.ANY),
            pl.BlockSpec(memory_space=pltpu.VMEM),
        ],
        out_specs=pl.BlockSpec(memory_space=pltpu.VMEM),
        scratch_shapes=[
            pltpu.VMEM((3, 2, S_LOC, H * D), BF),
            pltpu.VMEM((1, S_LOC, H, D), jnp.float32),
            pltpu.VMEM((S_LOC, H * D), BF),
            pltpu.VMEM((S_LOC, H * D), jnp.float32),
            pltpu.VMEM((S_LOC, H), jnp.float32),
            pltpu.SemaphoreType.DMA,
            pltpu.SemaphoreType.DMA((N_DEV - 1,)),
            pltpu.SemaphoreType.DMA((N_DEV - 1,)),
        ],
        compiler_params=pltpu.CompilerParams(
            collective_id=0,
            vmem_limit_bytes=100 * 1024 * 1024,
        ),
    )(x, Wq, K_ext, V_ext, Wo)


# device time: 114890 ns/iter; 1.6175x vs baseline; 1.6175x over previous
import jax
import jax.numpy as jnp
from jax import lax
from jax.experimental import pallas as pl
from jax.experimental.pallas import tpu as pltpu

N_DEV = 4
S_LOC = 1024
H = 8
D = 128
BLK = 64
SCALE = 0.08838834764831843
BF = jnp.bfloat16


def kernel(x, Wq, K_ext, V_ext, Wo):
    def body(x_ref, wq_ref, k_ref, v_ref, wo_ref, out_ref,
             kv_scr, stage, q_scr, acc_scr, wsum_scr,
             copy_sem, send_r, recv_r, send_l, recv_l):
        my = lax.axis_index("i")
        left = (my - 1) % N_DEV
        right = (my + 1) % N_DEV

        cp = pltpu.make_async_copy(k_ref, stage, copy_sem)
        cp.start()
        cp.wait()
        kv_scr[0, 0] = jnp.reshape(stage[0], (S_LOC, H * D)).astype(BF)
        cp = pltpu.make_async_copy(v_ref, stage, copy_sem)
        cp.start()
        cp.wait()
        kv_scr[0, 1] = jnp.reshape(stage[0], (S_LOC, H * D)).astype(BF)

        barrier_sem = pltpu.get_barrier_semaphore()
        for nbr in (left, right):
            pl.semaphore_signal(barrier_sem, inc=1, device_id=(nbr,),
                                device_id_type=pl.DeviceIdType.MESH)
        pl.semaphore_wait(barrier_sem, 2)

        r0 = pltpu.make_async_remote_copy(
            src_ref=kv_scr.at[0], dst_ref=kv_scr.at[1],
            send_sem=send_r.at[0], recv_sem=recv_r.at[0],
            device_id=(right,), device_id_type=pl.DeviceIdType.MESH)
        r0.start()
        l0 = pltpu.make_async_remote_copy(
            src_ref=kv_scr.at[0], dst_ref=kv_scr.at[3],
            send_sem=send_l.at[0], recv_sem=recv_l.at[0],
            device_id=(left,), device_id_type=pl.DeviceIdType.MESH)
        l0.start()

        q_scr[...] = (jnp.dot(x_ref[0].astype(BF), wq_ref[...].astype(BF),
                              preferred_element_type=jnp.float32)
                      * SCALE).astype(BF)

        acc_scr[...] = jnp.zeros((S_LOC, H * D), jnp.float32)
        wsum_scr[...] = jnp.zeros((S_LOC, H), jnp.float32)

        R = S_LOC // 2

        def fold_chunk(slot):
            origin = (my - slot) % N_DEV
            kb = (origin * S_LOC
                  + lax.broadcasted_iota(jnp.int32, (1, S_LOC), 1)) // BLK
            for r in range(S_LOC // R):
                rs = slice(r * R, (r + 1) * R)
                qb = (my * S_LOC + r * R
                      + lax.broadcasted_iota(jnp.int32, (R, 1), 0)) // BLK
                mask = (qb == kb) | (kb == 0) | ((qb + kb) % 3 == 0)
                for hd in range(H):
                    sl = slice(hd * D, (hd + 1) * D)
                    s = lax.dot_general(q_scr[rs, sl], kv_scr[slot, 0, :, sl],
                                        (((1,), (1,)), ((), ())),
                                        preferred_element_type=jnp.float32)
                    w = jnp.where(mask, jnp.exp(s), 0.0)
                    wsum_scr[rs, hd:hd + 1] = (
                        wsum_scr[rs, hd:hd + 1]
                        + jnp.sum(w, axis=1, keepdims=True))
                    acc_scr[rs, sl] = acc_scr[rs, sl] + lax.dot_general(
                        w.astype(BF), kv_scr[slot, 1, :, sl],
                        (((1,), (0,)), ((), ())),
                        preferred_element_type=jnp.float32)

        fold_chunk(0)

        r0.wait()
        l0.wait()

        r1 = pltpu.make_async_remote_copy(
            src_ref=kv_scr.at[1, 0], dst_ref=kv_scr.at[2, 0],
            send_sem=send_r.at[1], recv_sem=recv_r.at[1],
            device_id=(right,), device_id_type=pl.DeviceIdType.MESH)
        r1.start()
        l1 = pltpu.make_async_remote_copy(
            src_ref=kv_scr.at[3, 1], dst_ref=kv_scr.at[2, 1],
            send_sem=send_l.at[1], recv_sem=recv_l.at[1],
            device_id=(left,), device_id_type=pl.DeviceIdType.MESH)
        l1.start()

        fold_chunk(1)
        fold_chunk(3)

        r1.wait()
        l1.wait()

        fold_chunk(2)

        out = jnp.zeros((S_LOC, H * D), jnp.float32)
        for hd in range(H):
            sl = slice(hd * D, (hd + 1) * D)
            ctx = (acc_scr[:, sl] / wsum_scr[:, hd:hd + 1]).astype(BF)
            out = out + jnp.dot(ctx, wo_ref[sl, :].astype(BF),
                                preferred_element_type=jnp.float32)
        out_ref[0] = out

    return pl.pallas_call(
        body,
        out_shape=jax.ShapeDtypeStruct((1, S_LOC, H * D), jnp.float32),
        in_specs=[
            pl.BlockSpec(memory_space=pltpu.VMEM),
            pl.BlockSpec(memory_space=pltpu.VMEM),
            pl.BlockSpec(memory_space=pl.ANY),
            pl.BlockSpec(memory_space=pl.ANY),
            pl.BlockSpec(memory_space=pltpu.VMEM),
        ],
        out_specs=pl.BlockSpec(memory_space=pltpu.VMEM),
        scratch_shapes=[
            pltpu.VMEM((N_DEV, 2, S_LOC, H * D), BF),
            pltpu.VMEM((1, S_LOC, H, D), jnp.float32),
            pltpu.VMEM((S_LOC, H * D), BF),
            pltpu.VMEM((S_LOC, H * D), jnp.float32),
            pltpu.VMEM((S_LOC, H), jnp.float32),
            pltpu.SemaphoreType.DMA,
            pltpu.SemaphoreType.DMA((2,)),
            pltpu.SemaphoreType.DMA((2,)),
            pltpu.SemaphoreType.DMA((2,)),
            pltpu.SemaphoreType.DMA((2,)),
        ],
        compiler_params=pltpu.CompilerParams(
            collective_id=0,
            vmem_limit_bytes=100 * 1024 * 1024,
        ),
    )(x, Wq, K_ext, V_ext, Wo)
